# Initial kernel scaffold; baseline (speedup 1.0000x reference)
import jax
import jax.numpy as jnp
from jax.experimental import pallas as pl


def kernel(pr_boxes, gt_boxes, pr_class_logits, gt_classes):
    raise NotImplementedError("write your pallas kernel here")


# speedup vs baseline: 16.5853x; 16.5853x over previous
import functools

import jax
import jax.numpy as jnp
from jax.experimental import pallas as pl
from jax.experimental.pallas import tpu as pltpu

_N = 1000
_M = 100
_MP = 128
_B = 8
_BIG = 1 << 28
_EPS = 1e-7


def _giou_loss(x1, y1, x2, y2, xg1, yg1, xg2, yg2):
    xk1 = jnp.maximum(x1, xg1)
    yk1 = jnp.maximum(y1, yg1)
    xk2 = jnp.minimum(x2, xg2)
    yk2 = jnp.minimum(y2, yg2)
    valid = (yk2 > yk1) & (xk2 > xk1)
    inter = jnp.where(valid, (xk2 - xk1) * (yk2 - yk1), 0.0)
    union = (x2 - x1) * (y2 - y1) + (xg2 - xg1) * (yg2 - yg1) - inter
    iou = inter / (union + _EPS)
    xc1 = jnp.minimum(x1, xg1)
    yc1 = jnp.minimum(y1, yg1)
    xc2 = jnp.maximum(x2, xg2)
    yc2 = jnp.maximum(y2, yg2)
    area_c = (xc2 - xc1) * (yc2 - yc1)
    miou = iou - (area_c - union) / (area_c + _EPS)
    return 1.0 - miou


def _detr_body(pr_ref, gtr_ref, gtc_ref, out_ref, iou_ref):
    b = pl.program_id(0)

    prx1 = pr_ref[0, :, 0:1]
    pry1 = pr_ref[0, :, 1:2]
    prx2 = pr_ref[0, :, 2:3]
    pry2 = pr_ref[0, :, 3:4]

    gx1 = gtr_ref[0, 0:1, :]
    gy1 = gtr_ref[0, 1:2, :]
    gx2 = gtr_ref[0, 2:3, :]
    gy2 = gtr_ref[0, 3:4, :]

    riota = jax.lax.broadcasted_iota(jnp.int32, (_N, _MP), 0)
    ciota = jax.lax.broadcasted_iota(jnp.int32, (_N, _MP), 1)

    area1 = (prx2 - prx1) * (pry2 - pry1)
    area2 = (gx2 - gx1) * (gy2 - gy1)
    wx = jnp.maximum(jnp.minimum(prx2, gx2) - jnp.maximum(prx1, gx1), 0.0)
    wy = jnp.maximum(jnp.minimum(pry2, gy2) - jnp.maximum(pry1, gy1), 0.0)
    inter = wx * wy
    iou = inter / (area1 + area2 - inter)
    iou_ref[...] = jnp.where(ciota < _M, iou, -jnp.inf)

    lane = jax.lax.broadcasted_iota(jnp.int32, (1, _MP), 1)
    sub = jax.lax.broadcasted_iota(jnp.int32, (_MP, 1), 0)

    def step(k, carry):
        rows_row, rows_col, cols_col = carry
        m = iou_ref[...]
        v = jnp.max(m)
        flat = jnp.where(m == v, riota * _MP + ciota, _BIG)
        fm = jnp.min(flat)
        i = jax.lax.shift_right_logical(fm, 7)
        j = jax.lax.bitwise_and(fm, jnp.int32(_MP - 1))
        iou_ref[...] = jnp.where((riota == i) | (ciota == j), -jnp.inf, m)
        rows_row = jnp.where(lane == k, i, rows_row)
        rows_col = jnp.where(sub == k, i, rows_col)
        cols_col = jnp.where(sub == k, j, cols_col)
        return rows_row, rows_col, cols_col

    init = (jnp.full((1, _MP), _BIG, jnp.int32),
            jnp.full((_MP, 1), _BIG, jnp.int32),
            jnp.zeros((_MP, 1), jnp.int32))
    rows_row, rows_col, cols_col = jax.lax.fori_loop(0, _M, step, init)

    rank = jnp.sum(jnp.where(rows_col < rows_row, 1, 0), axis=0,
                   keepdims=True)
    gsel = jnp.sum(jnp.where(sub == rank, cols_col, 0), axis=0,
                   keepdims=True)

    pick = riota == rows_row
    ppx1 = jnp.sum(jnp.where(pick, prx1, 0.0), axis=0, keepdims=True)
    ppy1 = jnp.sum(jnp.where(pick, pry1, 0.0), axis=0, keepdims=True)
    ppx2 = jnp.sum(jnp.where(pick, prx2, 0.0), axis=0, keepdims=True)
    ppy2 = jnp.sum(jnp.where(pick, pry2, 0.0), axis=0, keepdims=True)
    matched = jnp.sum(jnp.where(pick, 1.0, 0.0), axis=1, keepdims=True)

    gcx1 = gtc_ref[0, :, 0:1]
    gcy1 = gtc_ref[0, :, 1:2]
    gcx2 = gtc_ref[0, :, 2:3]
    gcy2 = gtc_ref[0, :, 3:4]
    gpick = sub == gsel
    gpx1 = jnp.sum(jnp.where(gpick, gcx1, 0.0), axis=0, keepdims=True)
    gpy1 = jnp.sum(jnp.where(gpick, gcy1, 0.0), axis=0, keepdims=True)
    gpx2 = jnp.sum(jnp.where(gpick, gcx2, 0.0), axis=0, keepdims=True)
    gpy2 = jnp.sum(jnp.where(gpick, gcy2, 0.0), axis=0, keepdims=True)

    lpos = _giou_loss(ppx1, ppy1, ppx2, ppy2, gpx1, gpy1, gpx2, gpy2)
    pos_mean = jnp.sum(jnp.where(lane < _M, lpos, 0.0)) / _M

    zero = jnp.zeros_like(prx1)
    lneg = _giou_loss(prx1, pry1, prx2, pry2, zero, zero, zero, zero)
    neg_mean = jnp.sum(lneg * (1.0 - matched)) / (_N - _M)

    @pl.when(b == 0)
    def _():
        out_ref[...] = jnp.zeros_like(out_ref)

    out_ref[...] = out_ref[...] + (pos_mean + neg_mean)

    @pl.when(b == _B - 1)
    def _():
        out_ref[...] = out_ref[...] / (_B * 2.0)


@functools.partial(jax.jit, static_argnames=())
def kernel(pr_boxes, gt_boxes, pr_class_logits, gt_classes):
    del pr_class_logits, gt_classes
    gt_pad = jnp.pad(gt_boxes, ((0, 0), (0, _MP - _M), (0, 0)))
    gt_rows = jnp.transpose(gt_pad, (0, 2, 1))
    out = pl.pallas_call(
        _detr_body,
        grid=(_B,),
        in_specs=[
            pl.BlockSpec((1, _N, 4), lambda b: (b, 0, 0)),
            pl.BlockSpec((1, 4, _MP), lambda b: (b, 0, 0)),
            pl.BlockSpec((1, _MP, 4), lambda b: (b, 0, 0)),
        ],
        out_specs=pl.BlockSpec((1, 1), lambda b: (0, 0)),
        out_shape=jax.ShapeDtypeStruct((1, 1), jnp.float32),
        scratch_shapes=[pltpu.VMEM((_N, _MP), jnp.float32)],
    )(pr_boxes, gt_rows, gt_pad)
    return out[0, 0]
